# baseline (speedup 1.0000x reference)
import functools

import jax
import jax.numpy as jnp
from jax import lax
from jax.experimental import pallas as pl
from jax.experimental.pallas import tpu as pltpu
from jax.experimental.pallas import tpu_sc as plsc

B, D, T, H, W = 2, 768, 16, 24, 24
HW = H * W
THW = T * HW
ROWS = B * D
NW = 32
RPW = ROWS // NW
RPC = 2
CHW = RPC * THW
NCHUNK = RPW // RPC
NBUF = 6
LEAD = 4
SCALE = float(D) ** -0.5
NCH = HW // 16


def _sc_body(x_hbm, tbl_hbm, out_hbm, ring, tbl_v, pe_v, ih_v, iw_v,
             sem_t, *sems):
    sem_l = list(sems[:NBUF])
    sem_s = list(sems[NBUF:])
    wid = lax.axis_index("s") * 2 + lax.axis_index("c")
    base_row = wid * RPW
    d0 = lax.rem(base_row, D)

    pltpu.make_async_copy(
        tbl_hbm.at[pl.ds(d0 * 64, RPW * 64)], tbl_v, sem_t).start()

    def _idx_body(c, carry):
        jv = lax.iota(jnp.int32, 16) + jnp.full((16,), c * 16, jnp.int32)
        c24 = jnp.full((16,), 24, jnp.int32)
        ih_v[pl.ds(c * 16, 16)] = (
            lax.div(jv, c24) + jnp.full((16,), 16, jnp.int32))
        iw_v[pl.ds(c * 16, 16)] = (
            lax.rem(jv, c24) + jnp.full((16,), 40, jnp.int32))
        return carry
    lax.fori_loop(0, NCH, _idx_body, 0)

    pltpu.make_async_copy(
        tbl_hbm.at[pl.ds(d0 * 64, RPW * 64)], tbl_v, sem_t).wait()

    def _chunk_src(k):
        return x_hbm.at[pl.ds((base_row + k * RPC) * THW, CHW)]

    def _chunk_dst(k):
        return out_hbm.at[pl.ds((base_row + k * RPC) * THW, CHW)]

    def _slot(b):
        return ring.at[pl.ds(b * CHW, CHW)]

    def _start_load(k, b):
        pltpu.make_async_copy(_chunk_src(k), _slot(b), sem_l[b]).start()

    def _wait_load(k, b):
        pltpu.make_async_copy(_chunk_src(k), _slot(b), sem_l[b]).wait()

    def _start_store(k, b):
        pltpu.make_async_copy(_slot(b), _chunk_dst(k), sem_s[b]).start()

    def _wait_store(k, b):
        pltpu.make_async_copy(_slot(b), _chunk_dst(k), sem_s[b]).wait()

    def _compute_chunk(k, b):
        buf = _slot(b)
        sclv = jnp.full((16,), SCALE, jnp.float32)
        for j in range(RPC):
            r = k * RPC + j
            rb = r * 64
            rbv = jnp.full((16,), rb, jnp.int32)

            def _pe_body(c, inner, _rbv=rbv):
                ih = ih_v[pl.ds(c * 16, 16)] + _rbv
                iw = iw_v[pl.ds(c * 16, 16)] + _rbv
                pe = (plsc.load_gather(tbl_v, [ih])
                      + plsc.load_gather(tbl_v, [iw]))
                pe_v[pl.ds(c * 16, 16)] = pe * sclv
                return inner
            lax.fori_loop(0, NCH, _pe_body, 0, unroll=2)

            etbs = [plsc.load_gather(
                        tbl_v, [jnp.full((16,), rb + t, jnp.int32)]) * sclv
                    for t in range(T)]
            joff = j * THW

            def _c_body(c, inner, _etbs=etbs, _joff=joff):
                c16 = c * 16
                pe_c = pe_v[pl.ds(c16, 16)]
                for t in range(T):
                    plsc.addupdate(buf.at[pl.ds(_joff + t * HW + c16, 16)],
                                   pe_c + _etbs[t])
                return inner
            lax.fori_loop(0, NCH, _c_body, 0)

    def _iteration(k, b, first):
        _wait_load(k, b)
        _compute_chunk(k, b)
        lead_slot = (b + LEAD) % NBUF
        if not first:
            _wait_store(k - (NBUF - LEAD), lead_slot)
        _start_store(k, b)
        nxt = jnp.minimum(k + LEAD, NCHUNK - 1)
        pltpu.make_async_copy(
            x_hbm.at[pl.ds((base_row + nxt * RPC) * THW, CHW)],
            _slot(lead_slot), sem_l[lead_slot]).start()

    for b in range(LEAD):
        _start_load(b, b)

    for k in range(NBUF):
        _iteration(k, k, first=(k < NBUF - LEAD))

    def _loop_body(g, carry):
        for b in range(NBUF):
            _iteration(g * NBUF + b, b, first=False)
        return carry
    lax.fori_loop(1, NCHUNK // NBUF, _loop_body, 0)

    for k in range(NCHUNK - (NBUF - LEAD), NCHUNK):
        _wait_store(k, k % NBUF)
    for k in range(NCHUNK - LEAD, NCHUNK):
        _wait_load(NCHUNK - 1, (k + LEAD) % NBUF)


_sc_call = functools.partial(
    pl.kernel,
    out_type=jax.ShapeDtypeStruct((ROWS * THW,), jnp.float32),
    mesh=plsc.VectorSubcoreMesh(core_axis_name="c", subcore_axis_name="s"),
    compiler_params=pltpu.CompilerParams(needs_layout_passes=False),
    scratch_types=(
        [
            pltpu.VMEM((NBUF * CHW,), jnp.float32),
            pltpu.VMEM((RPW * 64,), jnp.float32),
            pltpu.VMEM((HW,), jnp.float32),
            pltpu.VMEM((HW,), jnp.int32),
            pltpu.VMEM((HW,), jnp.int32),
            pltpu.SemaphoreType.DMA,
        ]
        + [pltpu.SemaphoreType.DMA] * NBUF
        + [pltpu.SemaphoreType.DMA] * NBUF
    ),
)(_sc_body)


def kernel(x, emb_t, emb_h, emb_w):
    tbl = jnp.concatenate([emb_t.T, emb_h.T, emb_w.T], axis=1)
    out = _sc_call(x.reshape(-1), tbl.reshape(-1))
    return out.reshape(B, D, T, H, W)


# speedup vs baseline: 4.7032x; 4.7032x over previous
import functools

import jax
import jax.numpy as jnp
from jax import lax
from jax.experimental import pallas as pl
from jax.experimental.pallas import tpu as pltpu
from jax.experimental.pallas import tpu_sc as plsc

B, D, T, H, W = 2, 768, 16, 24, 24
NW = 32
PROWS = H * W
RPC = 12
CHW = RPC * D
NCHUNK = PROWS // RPC
NBUF = 6
LEAD = 4
SCALE = float(D) ** -0.5
DCH = D // 16
EH_OFF = T * D
EW_OFF = (T + H) * D


def _sc_body(x_hbm, tbl_hbm, out_hbm, ring, et_v, eh_v, ew_v, sem_t, *sems):
    sem_l = list(sems[:NBUF])
    sem_s = list(sems[NBUF:])
    wid = lax.axis_index("s") * 2 + lax.axis_index("c")
    t_idx = lax.rem(wid, T)
    base = wid * PROWS * D

    cp_et = pltpu.make_async_copy(
        tbl_hbm.at[pl.ds(t_idx * D, D)], et_v, sem_t)
    cp_eh = pltpu.make_async_copy(
        tbl_hbm.at[pl.ds(EH_OFF, H * D)], eh_v, sem_t)
    cp_ew = pltpu.make_async_copy(
        tbl_hbm.at[pl.ds(EW_OFF, W * D)], ew_v, sem_t)
    cp_et.start()
    cp_eh.start()
    cp_ew.start()
    cp_et.wait()
    cp_eh.wait()
    cp_ew.wait()

    sclv = jnp.full((16,), SCALE, jnp.float32)

    def _scale_body(c, carry):
        ew_v[pl.ds(c * 16, 16)] = ew_v[pl.ds(c * 16, 16)] * sclv
        return carry
    lax.fori_loop(0, W * D // 16, _scale_body, 0, unroll=4)

    def _chunk_src(k):
        return x_hbm.at[pl.ds(base + k * CHW, CHW)]

    def _chunk_dst(k):
        return out_hbm.at[pl.ds(base + k * CHW, CHW)]

    def _slot(b):
        return ring.at[pl.ds(b * CHW, CHW)]

    def _start_load(k, b):
        pltpu.make_async_copy(_chunk_src(k), _slot(b), sem_l[b]).start()

    def _wait_load(k, b):
        pltpu.make_async_copy(_chunk_src(k), _slot(b), sem_l[b]).wait()

    def _start_store(k, b):
        pltpu.make_async_copy(_slot(b), _chunk_dst(k), sem_s[b]).start()

    def _wait_store(k, b):
        pltpu.make_async_copy(_slot(b), _chunk_dst(k), sem_s[b]).wait()

    def _compute_chunk(k, b):
        buf = _slot(b)
        hoff = (k // 2) * D
        woff = lax.rem(k, 2) * RPC * D

        def _dc_body(dc, carry):
            o = dc * 16
            pe_c = (et_v[pl.ds(o, 16)] + eh_v[pl.ds(hoff + o, 16)]) * sclv
            for j in range(RPC):
                acc = pe_c + ew_v[pl.ds(woff + j * D + o, 16)]
                plsc.addupdate(buf.at[pl.ds(j * D + o, 16)], acc)
            return carry
        lax.fori_loop(0, DCH, _dc_body, 0)

    def _iteration(k, b, first):
        _wait_load(k, b)
        _compute_chunk(k, b)
        lead_slot = (b + LEAD) % NBUF
        if not first:
            _wait_store(k - (NBUF - LEAD), lead_slot)
        _start_store(k, b)
        nxt = jnp.minimum(k + LEAD, NCHUNK - 1)
        pltpu.make_async_copy(
            x_hbm.at[pl.ds(base + nxt * CHW, CHW)],
            _slot(lead_slot), sem_l[lead_slot]).start()

    for b in range(LEAD):
        _start_load(b, b)

    for k in range(NBUF):
        _iteration(k, k, first=(k < NBUF - LEAD))

    def _loop_body(g, carry):
        for b in range(NBUF):
            _iteration(g * NBUF + b, b, first=False)
        return carry
    lax.fori_loop(1, NCHUNK // NBUF, _loop_body, 0)

    for k in range(NCHUNK - (NBUF - LEAD), NCHUNK):
        _wait_store(k, k % NBUF)
    for k in range(NCHUNK - LEAD, NCHUNK):
        _wait_load(NCHUNK - 1, (k + LEAD) % NBUF)


_sc_call = functools.partial(
    pl.kernel,
    out_type=jax.ShapeDtypeStruct((B * T * PROWS * D,), jnp.float32),
    mesh=plsc.VectorSubcoreMesh(core_axis_name="c", subcore_axis_name="s"),
    compiler_params=pltpu.CompilerParams(needs_layout_passes=False),
    scratch_types=(
        [
            pltpu.VMEM((NBUF * CHW,), jnp.float32),
            pltpu.VMEM((D,), jnp.float32),
            pltpu.VMEM((H * D,), jnp.float32),
            pltpu.VMEM((W * D,), jnp.float32),
            pltpu.SemaphoreType.DMA,
        ]
        + [pltpu.SemaphoreType.DMA] * NBUF
        + [pltpu.SemaphoreType.DMA] * NBUF
    ),
)(_sc_body)


def kernel(x, emb_t, emb_h, emb_w):
    xp = x.transpose(0, 2, 3, 4, 1).reshape(-1)
    tbl = jnp.concatenate([emb_t, emb_h, emb_w], axis=0)
    out = _sc_call(xp, tbl.reshape(-1))
    return out.reshape(B, T, H, W, D).transpose(0, 4, 1, 2, 3)
